# baseline (speedup 1.0000x reference)
import functools

import jax
import jax.numpy as jnp
from jax import lax
from jax.experimental import pallas as pl
from jax.experimental.pallas import tpu as pltpu
from jax.experimental.pallas import tpu_sc as plsc

D = 4
RANK = 2
HID = 128
OUT = 2 * D * RANK + D
TBLW = 24

NC, NS = 2, 16
NW = NC * NS

CHUNK = 512
GPC = CHUNK // 16

IDXB = 3200


def _one_table(xb, w2):
    xm = jnp.mean(xb.reshape(xb.shape[0] // D, D, HID), axis=1)
    px = jnp.dot(xm, w2, preferred_element_type=jnp.float32,
                 precision=lax.Precision.HIGHEST)
    qx = jnp.sum(xm * xm, axis=1, keepdims=True)
    colid = lax.broadcasted_iota(jnp.int32, (1, TBLW), 1)
    return jnp.where(colid == OUT + 1, qx, px)


def _tables_body(x_ref, e_ref, wgx_ref, wge_ref, outx_ref, oute_ref):
    outx_ref[...] = _one_table(x_ref[...], wgx_ref[...])
    oute_ref[...] = _one_table(e_ref[...], wge_ref[...])


def _build_tables(x, e, wgx_pad, wge_pad, n_rows, bn):
    return pl.pallas_call(
        _tables_body,
        grid=(n_rows // bn,),
        in_specs=[
            pl.BlockSpec((bn * D, HID), lambda i: (i, 0)),
            pl.BlockSpec((bn * D, HID), lambda i: (i, 0)),
            pl.BlockSpec((HID, TBLW), lambda i: (0, 0)),
            pl.BlockSpec((HID, TBLW), lambda i: (0, 0)),
        ],
        out_specs=[
            pl.BlockSpec((bn, TBLW), lambda i: (i, 0)),
            pl.BlockSpec((bn, TBLW), lambda i: (i, 0)),
        ],
        out_shape=[
            jax.ShapeDtypeStruct((n_rows, TBLW), jnp.float32),
            jax.ShapeDtypeStruct((n_rows, TBLW), jnp.float32),
        ],
    )(x, e, wgx_pad, wge_pad)


def _idx_body(rc8_ref, sel_ref, pat_ref, out_ref):
    rcf = (rc8_ref[...] * 4).astype(jnp.float32)
    h = jnp.dot(rcf, sel_ref[...], preferred_element_type=jnp.float32,
                precision=lax.Precision.HIGHEST).astype(jnp.int32)
    m = h.shape[0]
    h = (h.reshape(m // 8, 8, 128) + pat_ref[...][None]).reshape(m, 128)
    out_ref[...] = h


def _build_idx(rc8, sel, pat, nnz):
    bm = (2 * IDXB) // 8
    grid = (nnz // IDXB,)
    return pl.pallas_call(
        _idx_body,
        grid=grid,
        in_specs=[
            pl.BlockSpec((bm, 8), lambda i: (i, 0)),
            pl.BlockSpec((8, 128), lambda i: (0, 0)),
            pl.BlockSpec((8, 128), lambda i: (0, 0)),
        ],
        out_specs=pl.BlockSpec((bm, 128), lambda i: (i, 0)),
        out_shape=jax.ShapeDtypeStruct((2 * nnz // 8, 128), jnp.int32),
    )(rc8, sel, pat)


@functools.lru_cache(maxsize=None)
def _sc_kernel(nnz):
    nchunks = nnz // CHUNK
    trips = (nchunks + NW - 1) // NW
    mesh = plsc.VectorSubcoreMesh(core_axis_name="c", subcore_axis_name="s",
                                  num_cores=NC, num_subcores=NS)

    @functools.partial(
        pl.kernel,
        out_type=[jax.ShapeDtypeStruct((nnz * D * D,), jnp.float32)],
        mesh=mesh,
        compiler_params=pltpu.CompilerParams(needs_layout_passes=False,
                                             use_tc_tiling_on_sc=False),
        scratch_types=[
            [pltpu.VMEM((CHUNK // 128, 128), jnp.int32) for _ in range(3)],
            [pltpu.VMEM((CHUNK // 128, 128), jnp.int32) for _ in range(3)],
            [pltpu.VMEM((CHUNK, TBLW), jnp.float32) for _ in range(3)],
            pltpu.VMEM((CHUNK * 16,), jnp.float32),
            pltpu.VMEM((TBLW,), jnp.float32),
            [pltpu.SemaphoreType.DMA for _ in range(3)],
            [pltpu.SemaphoreType.DMA for _ in range(3)],
            pltpu.SemaphoreType.DMA,
        ],
    )
    def k(tx_hbm, te_hbm, hei_hbm, b2_hbm, attr_out,
          rowvs, colvs, xrs, av, b2v, semidx, semx, seme):
        wid = lax.axis_index("s") * NC + lax.axis_index("c")
        pltpu.sync_copy(b2_hbm, b2v)
        lane = lax.iota(jnp.int32, 16)
        b2s = [plsc.load_gather(b2v, [jnp.full((16,), j, dtype=jnp.int32)])
               for j in range(OUT)]

        def one_group(g, xr):
            ridx = g * 16 + lane

            def cfull(v):
                return jnp.full((16,), v, dtype=jnp.int32)

            sq = plsc.load_gather(xr, [ridx, cfull(OUT)])
            qq = plsc.load_gather(xr, [ridx, cfull(OUT + 1)])
            mu = sq * (1.0 / (2 * HID))
            var = qq * (1.0 / (2 * HID)) - mu * mu
            v = var + 1e-5
            ii = plsc.bitcast(v, jnp.int32)
            ii = jnp.int32(0x5F3759DF) - lax.shift_right_arithmetic(ii, 1)
            y = plsc.bitcast(ii, jnp.float32)
            for _ in range(3):
                y = y * (1.5 - 0.5 * v * y * y)
            yn = 0.0 - y
            s = []
            for j in range(OUT):
                uj = plsc.load_gather(xr, [ridx, cfull(j)])
                z = uj * yn - b2s[j]
                s.append(1.0 / (1.0 + jnp.exp(z)))
            pos0 = ridx * 16
            for i in range(D):
                for jj in range(D):
                    o = s[2 * i] * s[2 * D + 2 * jj] + s[2 * i + 1] * s[2 * D + 2 * jj + 1]
                    if i == jj:
                        o = o + s[4 * D + i]
                    plsc.store_scatter(av, [pos0 + (D * i + jj)], o)

        nsub = CHUNK // 128

        def issue_idx(c, s):
            pltpu.async_copy(hei_hbm.at[0, pl.ds(c * nsub, nsub), :], rowvs[s], semidx[s])
            pltpu.async_copy(hei_hbm.at[1, pl.ds(c * nsub, nsub), :], colvs[s], semidx[s])

        def wait_idx(c, s):
            pltpu.make_async_copy(
                hei_hbm.at[0, pl.ds(c * nsub, nsub), :], rowvs[s], semidx[s]).wait()
            pltpu.make_async_copy(
                hei_hbm.at[1, pl.ds(c * nsub, nsub), :], colvs[s], semidx[s]).wait()

        def issue_x(s):
            for kk in range(nsub):
                pltpu.async_copy(tx_hbm.at[rowvs[s].at[kk]],
                                 xrs[s].at[pl.ds(kk * 128, 128), :], semx[s])

        def wait_x(s):
            for kk in range(nsub):
                pltpu.make_async_copy(tx_hbm.at[rowvs[s].at[kk]],
                                      xrs[s].at[pl.ds(kk * 128, 128), :], semx[s]).wait()

        def issue_e(s):
            for kk in range(nsub):
                pltpu.async_copy(te_hbm.at[colvs[s].at[kk]],
                                 xrs[s].at[pl.ds(kk * 128, 128), :], seme, add=True)

        def wait_e(s):
            for kk in range(nsub):
                pltpu.make_async_copy(te_hbm.at[colvs[s].at[kk]],
                                      xrs[s].at[pl.ds(kk * 128, 128), :], seme).wait()

        def compute(c, s):
            def group_body(gi, carry2):
                one_group(gi * 2, xrs[s])
                one_group(gi * 2 + 1, xrs[s])
                return carry2

            lax.fori_loop(0, GPC // 2, group_body, 0)
            pltpu.sync_copy(av, attr_out.at[pl.ds(c * (CHUNK * 16), CHUNK * 16)])

        def pipe_step(t, s):
            s1, s2 = (s + 1) % 3, (s + 2) % 3
            c = wid + t * NW
            c1, c2, c3 = c + NW, c + 2 * NW, c + 3 * NW

            @pl.when(c < nchunks)
            def _():
                wait_e(s)

            @pl.when(c3 < nchunks)
            def _():
                issue_idx(c3, s)

            @pl.when(c2 < nchunks)
            def _():
                wait_idx(c2, s2)
                issue_x(s2)

            @pl.when(c1 < nchunks)
            def _():
                wait_x(s1)
                issue_e(s1)

            @pl.when(c < nchunks)
            def _():
                compute(c, s)

        pltpu.sync_copy(hei_hbm.at[0, pl.ds(wid * nsub, nsub), :], rowvs[0])
        pltpu.sync_copy(hei_hbm.at[1, pl.ds(wid * nsub, nsub), :], colvs[0])
        pltpu.sync_copy(hei_hbm.at[0, pl.ds((wid + NW) * nsub, nsub), :], rowvs[1])
        pltpu.sync_copy(hei_hbm.at[1, pl.ds((wid + NW) * nsub, nsub), :], colvs[1])
        issue_x(0)
        wait_x(0)
        issue_e(0)
        issue_idx(wid + 2 * NW, 2)
        issue_x(1)

        def loop_body(u, carry):
            t = u * 3
            pipe_step(t, 0)
            pipe_step(t + 1, 1)
            pipe_step(t + 2, 2)
            return carry

        lax.fori_loop(0, (trips + 3) // 3, loop_body, 0)

    return k


def kernel(x, e, hyperedge_index, node_types, hyperedge_types, ln_gamma, ln_beta, W, b):
    f32 = jnp.float32
    n_nodes = x.shape[0] // D
    n_edges = e.shape[0] // D
    nnz = hyperedge_index.shape[1]

    Wg = W * ln_gamma[:, None]
    S = jnp.sum(Wg, axis=0)
    b2 = b + ln_beta @ W
    wg_x = jnp.zeros((HID, TBLW), f32) \
        .at[:, :OUT].set(Wg[:HID] - S[None, :] / (2.0 * HID)) \
        .at[:, OUT].set(1.0)
    wg_e = jnp.zeros((HID, TBLW), f32) \
        .at[:, :OUT].set(Wg[HID:] - S[None, :] / (2.0 * HID)) \
        .at[:, OUT].set(1.0)
    b2p = jnp.zeros((TBLW,), f32).at[:OUT].set(b2)

    tab_x, tab_e = _build_tables(x, e, wg_x, wg_e, n_nodes, 2000)

    hei3 = hyperedge_index.reshape(2, nnz // 128, 128)
    attr_out, = _sc_kernel(nnz)(tab_x, tab_e, hei3, b2p)

    sel = (lax.broadcasted_iota(jnp.int32, (8, 128), 1) // 16
           == lax.broadcasted_iota(jnp.int32, (8, 128), 0)).astype(f32)
    lanes8 = lax.broadcasted_iota(jnp.int32, (8, 128), 1)
    rows8i = lax.broadcasted_iota(jnp.int32, (8, 128), 0)
    pat = jnp.where(rows8i % 2 == 0,
                    lax.shift_right_logical(lax.bitwise_and(lanes8, 15), 2),
                    lax.bitwise_and(lanes8, 3))
    rc8 = hyperedge_index.reshape(2, nnz // 8, 8).transpose(1, 0, 2).reshape(nnz // 4, 8)
    idx_pairs = _build_idx(rc8, sel, pat, nnz)
    n16 = nnz * D * D
    idx_out = jnp.swapaxes(idx_pairs.reshape(n16 // 128, 2, 128), 0, 1).reshape(2, n16)
    return idx_out, attr_out


# speedup vs baseline: 1.3645x; 1.3645x over previous
import functools

import jax
import jax.numpy as jnp
from jax import lax
from jax.experimental import pallas as pl
from jax.experimental.pallas import tpu as pltpu
from jax.experimental.pallas import tpu_sc as plsc

D = 4
RANK = 2
HID = 128
OUT = 2 * D * RANK + D
TBLW = 24

NC, NS = 2, 16
NW = NC * NS

CHUNK = 512
GPC = CHUNK // 16

IDXB = 3200


def _one_table(xb, w2):
    xm = jnp.mean(xb.reshape(xb.shape[0] // D, D, HID), axis=1)
    px = jnp.dot(xm, w2, preferred_element_type=jnp.float32,
                 precision=lax.Precision.HIGHEST)
    qx = jnp.sum(xm * xm, axis=1, keepdims=True)
    colid = lax.broadcasted_iota(jnp.int32, (1, TBLW), 1)
    return jnp.where(colid == OUT + 1, qx, px)


def _tables_body(x_ref, e_ref, wgx_ref, wge_ref, outx_ref, oute_ref):
    outx_ref[...] = _one_table(x_ref[...], wgx_ref[...])
    oute_ref[...] = _one_table(e_ref[...], wge_ref[...])


def _build_tables(x, e, wgx_pad, wge_pad, n_rows, bn):
    return pl.pallas_call(
        _tables_body,
        grid=(n_rows // bn,),
        in_specs=[
            pl.BlockSpec((bn * D, HID), lambda i: (i, 0)),
            pl.BlockSpec((bn * D, HID), lambda i: (i, 0)),
            pl.BlockSpec((HID, TBLW), lambda i: (0, 0)),
            pl.BlockSpec((HID, TBLW), lambda i: (0, 0)),
        ],
        out_specs=[
            pl.BlockSpec((bn, TBLW), lambda i: (i, 0)),
            pl.BlockSpec((bn, TBLW), lambda i: (i, 0)),
        ],
        out_shape=[
            jax.ShapeDtypeStruct((n_rows, TBLW), jnp.float32),
            jax.ShapeDtypeStruct((n_rows, TBLW), jnp.float32),
        ],
    )(x, e, wgx_pad, wge_pad)


def _idx_body(r8_ref, c8_ref, sel_ref, out_ref):
    lanes = lax.broadcasted_iota(jnp.int32, (1, 128), 1)
    pat0 = lax.shift_right_logical(lax.bitwise_and(lanes, 15), 2)
    pat1 = lax.bitwise_and(lanes, 3)
    sel = sel_ref[...]
    rf = (r8_ref[0] * 4).astype(jnp.float32)
    cf = (c8_ref[0] * 4).astype(jnp.float32)
    h0 = jnp.dot(rf, sel, preferred_element_type=jnp.float32,
                 precision=lax.Precision.HIGHEST).astype(jnp.int32) + pat0
    h1 = jnp.dot(cf, sel, preferred_element_type=jnp.float32,
                 precision=lax.Precision.HIGHEST).astype(jnp.int32) + pat1
    m = h0.shape[0]
    out_ref[...] = jnp.concatenate(
        [h0.reshape(m, 1, 128), h1.reshape(m, 1, 128)], axis=1
    ).reshape(2 * m, 128)


def _build_idx(hei28, sel, nnz):
    bm = IDXB // 8
    grid = (nnz // IDXB,)
    return pl.pallas_call(
        _idx_body,
        grid=grid,
        in_specs=[
            pl.BlockSpec((1, bm, 8), lambda i: (0, i, 0)),
            pl.BlockSpec((1, bm, 8), lambda i: (1, i, 0)),
            pl.BlockSpec((8, 128), lambda i: (0, 0)),
        ],
        out_specs=pl.BlockSpec((2 * bm, 128), lambda i: (i, 0)),
        out_shape=jax.ShapeDtypeStruct((2 * nnz // 8, 128), jnp.int32),
    )(hei28, hei28, sel)


@functools.lru_cache(maxsize=None)
def _sc_kernel(nnz):
    nchunks = nnz // CHUNK
    trips = (nchunks + NW - 1) // NW
    mesh = plsc.VectorSubcoreMesh(core_axis_name="c", subcore_axis_name="s",
                                  num_cores=NC, num_subcores=NS)

    @functools.partial(
        pl.kernel,
        out_type=[jax.ShapeDtypeStruct((nnz * D * D,), jnp.float32)],
        mesh=mesh,
        compiler_params=pltpu.CompilerParams(needs_layout_passes=False,
                                             use_tc_tiling_on_sc=False),
        scratch_types=[
            [pltpu.VMEM((CHUNK // 128, 128), jnp.int32) for _ in range(3)],
            [pltpu.VMEM((CHUNK // 128, 128), jnp.int32) for _ in range(3)],
            [pltpu.VMEM((CHUNK, TBLW), jnp.float32) for _ in range(3)],
            pltpu.VMEM((CHUNK * 16,), jnp.float32),
            pltpu.VMEM((TBLW,), jnp.float32),
            [pltpu.SemaphoreType.DMA for _ in range(3)],
            [pltpu.SemaphoreType.DMA for _ in range(3)],
            pltpu.SemaphoreType.DMA,
        ],
    )
    def k(tx_hbm, te_hbm, hei_hbm, b2_hbm, attr_out,
          rowvs, colvs, xrs, av, b2v, semidx, semx, seme):
        wid = lax.axis_index("s") * NC + lax.axis_index("c")
        pltpu.sync_copy(b2_hbm, b2v)
        lane = lax.iota(jnp.int32, 16)
        b2s = [plsc.load_gather(b2v, [jnp.full((16,), j, dtype=jnp.int32)])
               for j in range(OUT)]

        def one_group(g, xr):
            ridx = g * 16 + lane

            def cfull(v):
                return jnp.full((16,), v, dtype=jnp.int32)

            sq = plsc.load_gather(xr, [ridx, cfull(OUT)])
            qq = plsc.load_gather(xr, [ridx, cfull(OUT + 1)])
            mu = sq * (1.0 / (2 * HID))
            var = qq * (1.0 / (2 * HID)) - mu * mu
            v = var + 1e-5
            ii = plsc.bitcast(v, jnp.int32)
            ii = jnp.int32(0x5F3759DF) - lax.shift_right_arithmetic(ii, 1)
            y = plsc.bitcast(ii, jnp.float32)
            for _ in range(3):
                y = y * (1.5 - 0.5 * v * y * y)
            yn = 0.0 - y
            s = []
            for j in range(OUT):
                uj = plsc.load_gather(xr, [ridx, cfull(j)])
                z = uj * yn - b2s[j]
                s.append(1.0 / (1.0 + jnp.exp(z)))
            pos0 = ridx * 16
            for i in range(D):
                for jj in range(D):
                    o = s[2 * i] * s[2 * D + 2 * jj] + s[2 * i + 1] * s[2 * D + 2 * jj + 1]
                    if i == jj:
                        o = o + s[4 * D + i]
                    plsc.store_scatter(av, [pos0 + (D * i + jj)], o)

        nsub = CHUNK // 128

        def issue_idx(c, s):
            pltpu.async_copy(hei_hbm.at[0, pl.ds(c * nsub, nsub), :], rowvs[s], semidx[s])
            pltpu.async_copy(hei_hbm.at[1, pl.ds(c * nsub, nsub), :], colvs[s], semidx[s])

        def wait_idx(c, s):
            pltpu.make_async_copy(
                hei_hbm.at[0, pl.ds(c * nsub, nsub), :], rowvs[s], semidx[s]).wait()
            pltpu.make_async_copy(
                hei_hbm.at[1, pl.ds(c * nsub, nsub), :], colvs[s], semidx[s]).wait()

        def issue_x(s):
            for kk in range(nsub):
                pltpu.async_copy(tx_hbm.at[rowvs[s].at[kk]],
                                 xrs[s].at[pl.ds(kk * 128, 128), :], semx[s])

        def wait_x(s):
            for kk in range(nsub):
                pltpu.make_async_copy(tx_hbm.at[rowvs[s].at[kk]],
                                      xrs[s].at[pl.ds(kk * 128, 128), :], semx[s]).wait()

        def issue_e(s):
            for kk in range(nsub):
                pltpu.async_copy(te_hbm.at[colvs[s].at[kk]],
                                 xrs[s].at[pl.ds(kk * 128, 128), :], seme, add=True)

        def wait_e(s):
            for kk in range(nsub):
                pltpu.make_async_copy(te_hbm.at[colvs[s].at[kk]],
                                      xrs[s].at[pl.ds(kk * 128, 128), :], seme).wait()

        def compute(c, s):
            def group_body(gi, carry2):
                one_group(gi * 2, xrs[s])
                one_group(gi * 2 + 1, xrs[s])
                return carry2

            lax.fori_loop(0, GPC // 2, group_body, 0)
            pltpu.sync_copy(av, attr_out.at[pl.ds(c * (CHUNK * 16), CHUNK * 16)])

        def pipe_step(t, s):
            s1, s2 = (s + 1) % 3, (s + 2) % 3
            c = wid + t * NW
            c1, c2, c3 = c + NW, c + 2 * NW, c + 3 * NW

            @pl.when(c < nchunks)
            def _():
                wait_e(s)

            @pl.when(c3 < nchunks)
            def _():
                issue_idx(c3, s)

            @pl.when(c2 < nchunks)
            def _():
                wait_idx(c2, s2)
                issue_x(s2)

            @pl.when(c1 < nchunks)
            def _():
                wait_x(s1)
                issue_e(s1)

            @pl.when(c < nchunks)
            def _():
                compute(c, s)

        pltpu.sync_copy(hei_hbm.at[0, pl.ds(wid * nsub, nsub), :], rowvs[0])
        pltpu.sync_copy(hei_hbm.at[1, pl.ds(wid * nsub, nsub), :], colvs[0])
        pltpu.sync_copy(hei_hbm.at[0, pl.ds((wid + NW) * nsub, nsub), :], rowvs[1])
        pltpu.sync_copy(hei_hbm.at[1, pl.ds((wid + NW) * nsub, nsub), :], colvs[1])
        issue_x(0)
        wait_x(0)
        issue_e(0)
        issue_idx(wid + 2 * NW, 2)
        issue_x(1)

        def loop_body(u, carry):
            t = u * 3
            pipe_step(t, 0)
            pipe_step(t + 1, 1)
            pipe_step(t + 2, 2)
            return carry

        lax.fori_loop(0, (trips + 3) // 3, loop_body, 0)

    return k


def kernel(x, e, hyperedge_index, node_types, hyperedge_types, ln_gamma, ln_beta, W, b):
    f32 = jnp.float32
    n_nodes = x.shape[0] // D
    n_edges = e.shape[0] // D
    nnz = hyperedge_index.shape[1]

    Wg = W * ln_gamma[:, None]
    S = jnp.sum(Wg, axis=0)
    b2 = b + ln_beta @ W
    wg_x = jnp.zeros((HID, TBLW), f32) \
        .at[:, :OUT].set(Wg[:HID] - S[None, :] / (2.0 * HID)) \
        .at[:, OUT].set(1.0)
    wg_e = jnp.zeros((HID, TBLW), f32) \
        .at[:, :OUT].set(Wg[HID:] - S[None, :] / (2.0 * HID)) \
        .at[:, OUT].set(1.0)
    b2p = jnp.zeros((TBLW,), f32).at[:OUT].set(b2)

    tab_x, tab_e = _build_tables(x, e, wg_x, wg_e, n_nodes, 2000)

    hei3 = hyperedge_index.reshape(2, nnz // 128, 128)
    attr_out, = _sc_kernel(nnz)(tab_x, tab_e, hei3, b2p)

    sel = (lax.broadcasted_iota(jnp.int32, (8, 128), 1) // 16
           == lax.broadcasted_iota(jnp.int32, (8, 128), 0)).astype(f32)
    hei28 = hyperedge_index.reshape(2, nnz // 8, 8)
    idx_pairs = _build_idx(hei28, sel, nnz)
    n16 = nnz * D * D
    idx_out = jnp.swapaxes(idx_pairs.reshape(n16 // 128, 2, 128), 0, 1).reshape(2, n16)
    return idx_out, attr_out
